# baseline (speedup 1.0000x reference)
import functools

import jax
import jax.numpy as jnp
from jax import lax
from jax.experimental import pallas as pl
from jax.experimental.pallas import tpu as pltpu
from jax.experimental.pallas import tpu_sc as plsc

_SEQ = 8192
_W = 32
_DIM = _W * _W


_NBUF = 4
_CHUNK = 16
_DEFER = 2


def _copy_body(table_hbm, out_hbm, *scratch):
    bufs = scratch[:_NBUF]
    sin = scratch[_NBUF:2 * _NBUF]
    sout = scratch[2 * _NBUF:]
    info = plsc.get_sparse_core_info()
    nw = info.num_cores * info.num_subcores
    rows = _SEQ // nw
    nchunks = rows // _CHUNK
    wid = lax.axis_index("s") * info.num_cores + lax.axis_index("c")
    base = wid * rows

    def in_copy(b, c):
        return pltpu.make_async_copy(
            table_hbm.at[pl.ds(base + c * _CHUNK, _CHUNK)], bufs[b], sin[b])

    def out_copy(b, c):
        return pltpu.make_async_copy(
            bufs[b], out_hbm.at[pl.ds(base + c * _CHUNK, _CHUNK)], sout[b])

    for b in range(_NBUF):
        in_copy(b, b).start()
    for c in range(nchunks):
        b = c % _NBUF
        in_copy(b, c).wait()
        out_copy(b, c).start()
        j = c - _DEFER + _NBUF
        if c >= _DEFER and j < nchunks:
            bb = j % _NBUF
            out_copy(bb, j - _NBUF).wait()
            in_copy(bb, j).start()
    for c in range(max(0, nchunks - _NBUF), nchunks):
        out_copy(c % _NBUF, c).wait()


def kernel(x, position_embeddings):
    del x
    mesh = plsc.VectorSubcoreMesh(core_axis_name="c", subcore_axis_name="s")
    copy = functools.partial(
        pl.kernel,
        mesh=mesh,
        out_type=jax.ShapeDtypeStruct((_SEQ, _DIM), jnp.float32),
        scratch_types=(
            [pltpu.VMEM((_CHUNK, _DIM), jnp.float32) for _ in range(_NBUF)]
            + [pltpu.SemaphoreType.DMA for _ in range(2 * _NBUF)]
        ),
    )(_copy_body)
    out = copy(position_embeddings)
    return out.reshape(1, _SEQ, _W, _W)


# speedup vs baseline: 1.3858x; 1.3858x over previous
import jax
import jax.numpy as jnp
from jax.experimental import pallas as pl

_SEQ = 8192
_W = 32
_DIM = _W * _W
_BLK = 1024


def _tc_body(in_ref, out_ref):
    out_ref[...] = in_ref[...]


def kernel(x, position_embeddings):
    del x
    out = pl.pallas_call(
        _tc_body,
        grid=(_SEQ // _BLK,),
        in_specs=[pl.BlockSpec((_BLK, _DIM), lambda i: (i, 0))],
        out_specs=pl.BlockSpec((_BLK, _DIM), lambda i: (i, 0)),
        out_shape=jax.ShapeDtypeStruct((_SEQ, _DIM), jnp.float32),
    )(position_embeddings)
    return out.reshape(1, _SEQ, _W, _W)
